# baseline (speedup 1.0000x reference)
import functools

import jax
import jax.numpy as jnp
from jax import lax
from jax.experimental import pallas as pl
from jax.experimental.pallas import tpu as pltpu
from jax.experimental.pallas import tpu_sc as plsc

N = 100000
D = 32
F = 4 * D
NW = 32
B_PER_W = 3200
N_PAD = NW * B_PER_W
CB = 400
CHUNKS_PW = B_PER_W // CB
CBF = CB * F
TSIZES = (20 * D, 10 * D, 25 * D, 10 * D)
TOFF = (0, TSIZES[0], TSIZES[0] + TSIZES[1], TSIZES[0] + TSIZES[1] + TSIZES[2])
TTOT = sum(TSIZES)


def _sc_embed(i0, i1, i2, i3, t0, t1, t2, t3):
    mesh = plsc.VectorSubcoreMesh(core_axis_name="c", subcore_axis_name="s")

    @functools.partial(
        pl.kernel,
        mesh=mesh,
        compiler_params=pltpu.CompilerParams(
            use_tc_tiling_on_sc=False, needs_layout_passes=False),
        out_type=jax.ShapeDtypeStruct((N_PAD * F,), jnp.float32),
        scratch_types=[
            pltpu.VMEM((4, B_PER_W), jnp.int32),
            pltpu.VMEM((TTOT,), jnp.float32),
            pltpu.VMEM((2 * CBF,), jnp.float32),
            pltpu.SemaphoreType.DMA,
        ],
    )
    def k(i0h, i1h, i2h, i3h, t0h, t1h, t2h, t3h, out, idx_v, tab_v, out_v,
          ssem):
        wid = lax.axis_index("s") * 2 + lax.axis_index("c")
        ab = wid * B_PER_W

        pltpu.sync_copy(i0h.at[pl.ds(ab, B_PER_W)], idx_v.at[0])
        pltpu.sync_copy(i1h.at[pl.ds(ab, B_PER_W)], idx_v.at[1])
        pltpu.sync_copy(i2h.at[pl.ds(ab, B_PER_W)], idx_v.at[2])
        pltpu.sync_copy(i3h.at[pl.ds(ab, B_PER_W)], idx_v.at[3])
        pltpu.sync_copy(t0h, tab_v.at[pl.ds(TOFF[0], TSIZES[0])])
        pltpu.sync_copy(t1h, tab_v.at[pl.ds(TOFF[1], TSIZES[1])])
        pltpu.sync_copy(t2h, tab_v.at[pl.ds(TOFF[2], TSIZES[2])])
        pltpu.sync_copy(t3h, tab_v.at[pl.ds(TOFF[3], TSIZES[3])])

        def chunk(q, carry):
            slot_base = lax.rem(q, 2) * CBF

            @pl.when(q >= 2)
            def _():
                pltpu.make_async_copy(
                    out_v.at[pl.ds(0, CBF)], out.at[pl.ds(0, CBF)], ssem
                ).wait()

            @plsc.parallel_loop(0, CB // 16, unroll=2)
            def group(g):
                off = q * CB + g * 16
                vis = [idx_v[t, pl.ds(off, 16)] * D for t in range(4)]
                for j in range(16):
                    ob = slot_base + (g * 16 + j) * F
                    for t in range(4):
                        base = TOFF[t] + pl.multiple_of(vis[t][j], D)
                        for h in range(2):
                            out_v[pl.ds(ob + t * D + h * 16, 16)] = (
                                tab_v[pl.ds(base + h * 16, 16)])

            pltpu.async_copy(
                out_v.at[pl.ds(slot_base, CBF)],
                out.at[pl.ds((ab + q * CB) * F, CBF)],
                ssem,
            )
            return carry

        lax.fori_loop(0, CHUNKS_PW, chunk, 0)
        pltpu.make_async_copy(out_v.at[pl.ds(0, CBF)], out.at[pl.ds(0, CBF)], ssem).wait()
        pltpu.make_async_copy(out_v.at[pl.ds(0, CBF)], out.at[pl.ds(0, CBF)], ssem).wait()

    return k(i0, i1, i2, i3, t0, t1, t2, t3)


def kernel(atom_type_index, atom_code_index, residue_code_index, residue_sequence_index,
           atom_type_table, atom_code_table, residue_code_table, residue_index_table):
    pad = N_PAD - N
    i0 = jnp.pad(atom_type_index, (0, pad))
    i1 = jnp.pad(atom_code_index, (0, pad))
    i2 = jnp.pad(residue_code_index, (0, pad))
    i3 = jnp.pad(residue_sequence_index, (0, pad))
    out = _sc_embed(i0, i1, i2, i3,
                    atom_type_table.reshape(-1), atom_code_table.reshape(-1),
                    residue_code_table.reshape(-1), residue_index_table.reshape(-1))
    return out.reshape(N_PAD, F)[:N]


# speedup vs baseline: 1.1354x; 1.1354x over previous
import functools

import jax
import jax.numpy as jnp
from jax import lax
from jax.experimental import pallas as pl
from jax.experimental.pallas import tpu as pltpu
from jax.experimental.pallas import tpu_sc as plsc

N = 100000
D = 32
F = 4 * D
NW = 32
B_PER_W = 3200
N_PAD = NW * B_PER_W
CB = 320
CHUNKS_PW = B_PER_W // CB
GROUPS = CB // 16
CBF = CB * F
TSIZES = (20 * D, 10 * D, 25 * D, 10 * D)
TOFF = (0, TSIZES[0], TSIZES[0] + TSIZES[1], TSIZES[0] + TSIZES[1] + TSIZES[2])
TTOT = sum(TSIZES)
T01 = 20 * 10 * 64
T23 = 25 * 10 * 64


def _sc_embed(i0, i1, i2, i3, t0, t1, t2, t3):
    mesh = plsc.VectorSubcoreMesh(core_axis_name="c", subcore_axis_name="s")

    @functools.partial(
        pl.kernel,
        mesh=mesh,
        compiler_params=pltpu.CompilerParams(
            use_tc_tiling_on_sc=False, needs_layout_passes=False),
        out_type=jax.ShapeDtypeStruct((N_PAD * F,), jnp.float32),
        scratch_types=[
            pltpu.VMEM((4, B_PER_W), jnp.int32),
            pltpu.VMEM((TTOT,), jnp.float32),
            pltpu.VMEM((T01 + T23,), jnp.float32),
            pltpu.VMEM((2 * CBF,), jnp.float32),
            pltpu.SemaphoreType.DMA,
        ],
    )
    def k(i0h, i1h, i2h, i3h, t0h, t1h, t2h, t3h, out, idx_v, tab_v, tabf_v,
          out_v, ssem):
        wid = lax.axis_index("s") * 2 + lax.axis_index("c")
        ab = wid * B_PER_W

        pltpu.sync_copy(i0h.at[pl.ds(ab, B_PER_W)], idx_v.at[0])
        pltpu.sync_copy(i1h.at[pl.ds(ab, B_PER_W)], idx_v.at[1])
        pltpu.sync_copy(i2h.at[pl.ds(ab, B_PER_W)], idx_v.at[2])
        pltpu.sync_copy(i3h.at[pl.ds(ab, B_PER_W)], idx_v.at[3])
        pltpu.sync_copy(t0h, tab_v.at[pl.ds(TOFF[0], TSIZES[0])])
        pltpu.sync_copy(t1h, tab_v.at[pl.ds(TOFF[1], TSIZES[1])])
        pltpu.sync_copy(t2h, tab_v.at[pl.ds(TOFF[2], TSIZES[2])])
        pltpu.sync_copy(t3h, tab_v.at[pl.ds(TOFF[3], TSIZES[3])])

        def build01(a, carry):
            s0 = pl.multiple_of(a * D, D)
            d0 = pl.multiple_of(a * (10 * 64), 64)
            rlo = tab_v[pl.ds(s0, 16)]
            rhi = tab_v[pl.ds(s0 + 16, 16)]
            for b in range(10):
                d = d0 + b * 64
                tabf_v[pl.ds(d, 16)] = rlo
                tabf_v[pl.ds(d + 16, 16)] = rhi
                s1 = TOFF[1] + b * D
                tabf_v[pl.ds(d + 32, 16)] = tab_v[pl.ds(s1, 16)]
                tabf_v[pl.ds(d + 48, 16)] = tab_v[pl.ds(s1 + 16, 16)]
            return carry

        lax.fori_loop(0, 20, build01, 0)

        def build23(c, carry):
            s2 = pl.multiple_of(TOFF[2] + c * D, D)
            d0 = pl.multiple_of(T01 + c * (10 * 64), 64)
            rlo = tab_v[pl.ds(s2, 16)]
            rhi = tab_v[pl.ds(s2 + 16, 16)]
            for b in range(10):
                d = d0 + b * 64
                tabf_v[pl.ds(d, 16)] = rlo
                tabf_v[pl.ds(d + 16, 16)] = rhi
                s3 = TOFF[3] + b * D
                tabf_v[pl.ds(d + 32, 16)] = tab_v[pl.ds(s3, 16)]
                tabf_v[pl.ds(d + 48, 16)] = tab_v[pl.ds(s3 + 16, 16)]
            return carry

        lax.fori_loop(0, 25, build23, 0)

        def chunk(q, carry):
            slot_base = lax.rem(q, 2) * CBF

            @pl.when(q >= 2)
            def _():
                pltpu.make_async_copy(
                    out_v.at[pl.ds(0, CBF)], out.at[pl.ds(0, CBF)], ssem
                ).wait()

            @plsc.parallel_loop(0, GROUPS)
            def group(g):
                off = q * CB + g * 16
                v01 = (idx_v[0, pl.ds(off, 16)] * 10
                       + idx_v[1, pl.ds(off, 16)]) * 64
                v23 = (idx_v[2, pl.ds(off, 16)] * 10
                       + idx_v[3, pl.ds(off, 16)]) * 64
                for j in range(16):
                    ob = slot_base + (g * 16 + j) * F
                    e01 = pl.multiple_of(v01[j], 64)
                    e23 = T01 + pl.multiple_of(v23[j], 64)
                    for h in range(4):
                        out_v[pl.ds(ob + h * 16, 16)] = (
                            tabf_v[pl.ds(e01 + h * 16, 16)])
                    for h in range(4):
                        out_v[pl.ds(ob + 64 + h * 16, 16)] = (
                            tabf_v[pl.ds(e23 + h * 16, 16)])

            pltpu.async_copy(
                out_v.at[pl.ds(slot_base, CBF)],
                out.at[pl.ds((ab + q * CB) * F, CBF)],
                ssem,
            )
            return carry

        lax.fori_loop(0, CHUNKS_PW, chunk, 0)
        pltpu.make_async_copy(out_v.at[pl.ds(0, CBF)], out.at[pl.ds(0, CBF)], ssem).wait()
        pltpu.make_async_copy(out_v.at[pl.ds(0, CBF)], out.at[pl.ds(0, CBF)], ssem).wait()

    return k(i0, i1, i2, i3, t0, t1, t2, t3)


def kernel(atom_type_index, atom_code_index, residue_code_index, residue_sequence_index,
           atom_type_table, atom_code_table, residue_code_table, residue_index_table):
    pad = N_PAD - N
    i0 = jnp.pad(atom_type_index, (0, pad))
    i1 = jnp.pad(atom_code_index, (0, pad))
    i2 = jnp.pad(residue_code_index, (0, pad))
    i3 = jnp.pad(residue_sequence_index, (0, pad))
    out = _sc_embed(i0, i1, i2, i3,
                    atom_type_table.reshape(-1), atom_code_table.reshape(-1),
                    residue_code_table.reshape(-1), residue_index_table.reshape(-1))
    return out.reshape(N_PAD, F)[:N]


# speedup vs baseline: 1.1931x; 1.0509x over previous
import functools

import jax
import jax.numpy as jnp
from jax import lax
from jax.experimental import pallas as pl
from jax.experimental.pallas import tpu as pltpu
from jax.experimental.pallas import tpu_sc as plsc

N = 100000
D = 32
F = 4 * D
NW = 32
B_PER_W = 3200
N_PAD = NW * B_PER_W
CB = 400
CHUNKS_PW = B_PER_W // CB
CBF = CB * F
TSIZES = (20 * D, 10 * D, 25 * D, 10 * D)
TOFF = (0, TSIZES[0], TSIZES[0] + TSIZES[1], TSIZES[0] + TSIZES[1] + TSIZES[2])
TTOT = sum(TSIZES)


def _sc_embed(i0, i1, i2, i3, t0, t1, t2, t3):
    mesh = plsc.VectorSubcoreMesh(core_axis_name="c", subcore_axis_name="s")

    @functools.partial(
        pl.kernel,
        mesh=mesh,
        compiler_params=pltpu.CompilerParams(
            use_tc_tiling_on_sc=False, needs_layout_passes=False),
        out_type=jax.ShapeDtypeStruct((N_PAD * F,), jnp.float32),
        scratch_types=[
            pltpu.VMEM((4, B_PER_W), jnp.int32),
            pltpu.VMEM((TTOT,), jnp.float32),
            pltpu.VMEM((2 * CBF,), jnp.float32),
            pltpu.SemaphoreType.DMA,
        ],
    )
    def k(i0h, i1h, i2h, i3h, t0h, t1h, t2h, t3h, out, idx_v, tab_v, out_v,
          ssem):
        wid = lax.axis_index("s") * 2 + lax.axis_index("c")
        ab = wid * B_PER_W

        pltpu.sync_copy(i0h.at[pl.ds(ab, B_PER_W)], idx_v.at[0])
        pltpu.sync_copy(i1h.at[pl.ds(ab, B_PER_W)], idx_v.at[1])
        pltpu.sync_copy(i2h.at[pl.ds(ab, B_PER_W)], idx_v.at[2])
        pltpu.sync_copy(i3h.at[pl.ds(ab, B_PER_W)], idx_v.at[3])
        pltpu.sync_copy(t0h, tab_v.at[pl.ds(TOFF[0], TSIZES[0])])
        pltpu.sync_copy(t1h, tab_v.at[pl.ds(TOFF[1], TSIZES[1])])
        pltpu.sync_copy(t2h, tab_v.at[pl.ds(TOFF[2], TSIZES[2])])
        pltpu.sync_copy(t3h, tab_v.at[pl.ds(TOFF[3], TSIZES[3])])

        def chunk(q, carry):
            slot_base = lax.rem(q, 2) * CBF

            @pl.when(q >= 2)
            def _():
                pltpu.make_async_copy(
                    out_v.at[pl.ds(0, CBF)], out.at[pl.ds(0, CBF)], ssem
                ).wait()

            @plsc.parallel_loop(0, CB // 16)
            def group(g):
                off = q * CB + g * 16
                vis = [idx_v[t, pl.ds(off, 16)] * D for t in range(4)]
                for j in range(16):
                    ob = slot_base + (g * 16 + j) * F
                    for t in range(4):
                        base = TOFF[t] + pl.multiple_of(vis[t][j], D)
                        for h in range(2):
                            out_v[pl.ds(ob + t * D + h * 16, 16)] = (
                                tab_v[pl.ds(base + h * 16, 16)])

            pltpu.async_copy(
                out_v.at[pl.ds(slot_base, CBF)],
                out.at[pl.ds((ab + q * CB) * F, CBF)],
                ssem,
            )
            return carry

        lax.fori_loop(0, CHUNKS_PW, chunk, 0)
        pltpu.make_async_copy(out_v.at[pl.ds(0, CBF)], out.at[pl.ds(0, CBF)], ssem).wait()
        pltpu.make_async_copy(out_v.at[pl.ds(0, CBF)], out.at[pl.ds(0, CBF)], ssem).wait()

    return k(i0, i1, i2, i3, t0, t1, t2, t3)


def kernel(atom_type_index, atom_code_index, residue_code_index, residue_sequence_index,
           atom_type_table, atom_code_table, residue_code_table, residue_index_table):
    pad = N_PAD - N
    i0 = jnp.pad(atom_type_index, (0, pad))
    i1 = jnp.pad(atom_code_index, (0, pad))
    i2 = jnp.pad(residue_code_index, (0, pad))
    i3 = jnp.pad(residue_sequence_index, (0, pad))
    out = _sc_embed(i0, i1, i2, i3,
                    atom_type_table.reshape(-1), atom_code_table.reshape(-1),
                    residue_code_table.reshape(-1), residue_index_table.reshape(-1))
    return out.reshape(N_PAD, F)[:N]
